# baseline (speedup 1.0000x reference)
import functools

import numpy as np
import jax
import jax.numpy as jnp
from jax import lax
from jax.experimental import pallas as pl
from jax.experimental.pallas import tpu as pltpu
from jax.experimental.pallas import tpu_sc as plsc

_THR = 0.18
_B, _C, _H, _W = 2, 96, 512, 512
_HR = _H // 2
_NSUB = 16
_RPT = _HR // _NSUB
_CPC = 4
_NCH = _C // _CPC
_IPC = _CPC * _RPT
_NV = _W // 16


def _build_indices():
    b = np.arange(_B)[:, None, None, None]
    sid = np.arange(_NSUB)[None, :, None, None]
    chunk = np.arange(_NCH)[None, None, :, None]
    t = np.arange(_IPC)[None, None, None, :]
    ch = chunk * _CPC + t // _RPT
    r = sid * _RPT + t % _RPT
    row_id = (b * _C + ch) * _H + 2 * r
    return jnp.asarray(row_id.astype(np.int32))


def _sc_body(table, idxs, mask_out, val_out,
             idx_v, buf0, buf1, acc, p_buf, mtile, vtile, shared, sem0, sem1):
    cid = lax.axis_index("c")
    sid = lax.axis_index("s")

    io = lax.iota(jnp.int32, 16)
    one = jnp.full((16,), 1.0, jnp.float32)
    zero = jnp.full((16,), 0.0, jnp.float32)
    even_f = jnp.where(io % 2 == 0, 1.0, 0.0)
    odd_f = one - even_f
    pat1_f = jnp.where(io % 4 == 1, 1.0, 0.0)
    pat3_f = jnp.where(io % 4 == 3, 1.0, 0.0)
    idx_m1 = (io + 15) % 16
    idx_p1 = (io + 1) % 16
    inv_c = jnp.float32(1.0 / _C)

    _gdn = lax.GatherDimensionNumbers(
        offset_dims=(), collapsed_slice_dims=(0,), start_index_map=(0,))

    def _take(v, idx):
        return lax.gather(v, idx[:, None], _gdn, (1,),
                          mode=lax.GatherScatterMode.PROMISE_IN_BOUNDS)

    pltpu.sync_copy(idxs.at[cid, sid], idx_v)

    bufs = (buf0, buf1)
    sems = (sem0, sem1)

    def gather(gi):
        return pltpu.make_async_copy(
            table.at[idx_v.at[gi]], bufs[gi % 2], sems[gi % 2])

    def accum_chunk(buf, first):
        @plsc.parallel_loop(0, _RPT * _NV, unroll=16)
        def posloop(pos):
            k = pos // _NV
            base = (pos % _NV) * 16
            v = buf[0 * _RPT + k, pl.ds(base, 16)]
            for ch in range(1, _CPC):
                v = v + buf[ch * _RPT + k, pl.ds(base, 16)]
            if first:
                acc[k, pl.ds(base, 16)] = v
            else:
                plsc.addupdate(acc.at[k, pl.ds(base, 16)], v)

    gather(0).start()
    for gi in range(_NCH):
        if gi + 1 < _NCH:
            gather(gi + 1).start()
        gather(gi).wait()
        accum_chunk(bufs[gi % 2], first=(gi == 0))

    def prow(k, _):
        rr = sid * _RPT + k
        par = (rr % 2).astype(jnp.float32)
        patt = pat1_f + (pat3_f - pat1_f) * par

        @plsc.parallel_loop(0, _NV, unroll=4)
        def pcol(j):
            base = j * 16
            hm = acc[k, pl.ds(base, 16)] * inv_c
            p_buf[pl.ds(k * _W + base, 16)] = jnp.where(hm <= _THR, patt, zero)
        return 0
    lax.fori_loop(0, _RPT, prow, 0)

    pltpu.sync_copy(p_buf.at[pl.ds(0, _W)], shared.at[sid])
    plsc.subcore_barrier()

    @pl.when(sid < _NSUB - 1)
    def _():
        pltpu.sync_copy(shared.at[sid + 1], p_buf.at[pl.ds(_RPT * _W, _W)])

    @pl.when(sid == _NSUB - 1)
    def _():
        @plsc.parallel_loop(0, _NV, unroll=4)
        def zcol(j):
            p_buf[pl.ds(_RPT * _W + j * 16, 16)] = zero

    def orow(k, _):
        def ocol(j, carry):
            prev, cur = carry
            base = j * 16
            flat = k * _W + base
            last = (j == _NV - 1).astype(jnp.float32)
            nxt = p_buf[pl.ds(flat + 16, 16)] * (1.0 - last)
            pdn = p_buf[pl.ds(flat + _W, 16)]
            psr = jnp.where(io == 0, _take(prev, idx_m1), _take(cur, idx_m1))
            psl = jnp.where(io == 15, _take(nxt, idx_p1), _take(cur, idx_p1))
            mtile[2 * k, pl.ds(base, 16)] = even_f * (one - psr)
            mtile[2 * k + 1, pl.ds(base, 16)] = odd_f * (one - jnp.maximum(cur, pdn))
            vtile[2 * k, pl.ds(base, 16)] = even_f * psl
            vtile[2 * k + 1, pl.ds(base, 16)] = zero
            return (cur, nxt)
        cur0 = p_buf[pl.ds(k * _W, 16)]
        lax.fori_loop(0, _NV, ocol, (zero, cur0), unroll=2)
        return 0
    lax.fori_loop(0, _RPT, orow, 0)

    rows = pl.ds(sid * 2 * _RPT, 2 * _RPT)
    pltpu.sync_copy(mtile, mask_out.at[cid, 0, rows, :])
    pltpu.sync_copy(vtile, val_out.at[cid, 0, rows, :])


@jax.jit
def kernel(sigma):
    table = sigma.reshape(_B * _C * _H, _W)
    idxs = _build_indices()
    out_sds = jax.ShapeDtypeStruct((_B, 1, _H, _W), jnp.float32)
    mesh = plsc.VectorSubcoreMesh(core_axis_name="c", subcore_axis_name="s")
    sc_fn = functools.partial(
        pl.kernel,
        mesh=mesh,
        out_type=[out_sds, out_sds],
        scratch_types=[
            pltpu.VMEM((_NCH, _IPC), jnp.int32),
            pltpu.VMEM((_IPC, _W), jnp.float32),
            pltpu.VMEM((_IPC, _W), jnp.float32),
            pltpu.VMEM((_RPT, _W), jnp.float32),
            pltpu.VMEM(((_RPT + 1) * _W,), jnp.float32),
            pltpu.VMEM((2 * _RPT, _W), jnp.float32),
            pltpu.VMEM((2 * _RPT, _W), jnp.float32),
            pltpu.VMEM_SHARED((_NSUB, _W), jnp.float32),
            pltpu.SemaphoreType.DMA,
            pltpu.SemaphoreType.DMA,
        ],
    )(_sc_body)
    mask, values = sc_fn(table, idxs)
    return mask, values


# speedup vs baseline: 1.0367x; 1.0367x over previous
import functools

import numpy as np
import jax
import jax.numpy as jnp
from jax import lax
from jax.experimental import pallas as pl
from jax.experimental.pallas import tpu as pltpu
from jax.experimental.pallas import tpu_sc as plsc

_THR = 0.18
_B, _C, _H, _W = 2, 96, 512, 512
_HR = _H // 2
_NSUB = 16
_RPT = _HR // _NSUB
_CPC = 4
_NCH = _C // _CPC
_IPC = _CPC * _RPT
_NV = _W // 16


def _build_indices():
    b = np.arange(_B)[:, None, None, None]
    sid = np.arange(_NSUB)[None, :, None, None]
    chunk = np.arange(_NCH)[None, None, :, None]
    t = np.arange(_IPC)[None, None, None, :]
    ch = chunk * _CPC + t // _RPT
    r = sid * _RPT + t % _RPT
    row_id = (b * _C + ch) * _H + 2 * r
    return jnp.asarray(row_id.astype(np.int32))


def _sc_body(table, idxs, mask_out, val_out,
             idx_v, buf0, buf1, acc, p_buf, mtile, vtile, shared, sem0, sem1):
    cid = lax.axis_index("c")
    sid = lax.axis_index("s")

    io = lax.iota(jnp.int32, 16)
    one = jnp.full((16,), 1.0, jnp.float32)
    zero = jnp.full((16,), 0.0, jnp.float32)
    even_f = jnp.where(io % 2 == 0, 1.0, 0.0)
    odd_f = one - even_f
    pat1_f = jnp.where(io % 4 == 1, 1.0, 0.0)
    pat3_f = jnp.where(io % 4 == 3, 1.0, 0.0)
    idx_m1 = (io + 15) % 16
    idx_p1 = (io + 1) % 16
    inv_c = jnp.float32(1.0 / _C)

    _gdn = lax.GatherDimensionNumbers(
        offset_dims=(), collapsed_slice_dims=(0,), start_index_map=(0,))

    def _take(v, idx):
        return lax.gather(v, idx[:, None], _gdn, (1,),
                          mode=lax.GatherScatterMode.PROMISE_IN_BOUNDS)

    pltpu.sync_copy(idxs.at[cid, sid], idx_v)

    bufs = (buf0, buf1)
    sems = (sem0, sem1)

    def gather(gi):
        return pltpu.make_async_copy(
            table.at[idx_v.at[gi]], bufs[gi % 2], sems[gi % 2])

    def accum_chunk(buf, first):
        @plsc.parallel_loop(0, _RPT * _NV, unroll=8)
        def posloop(pos):
            k = pos // _NV
            base = (pos % _NV) * 16
            v = buf[0 * _RPT + k, pl.ds(base, 16)]
            for ch in range(1, _CPC):
                v = v + buf[ch * _RPT + k, pl.ds(base, 16)]
            if first:
                acc[k, pl.ds(base, 16)] = v
            else:
                plsc.addupdate(acc.at[k, pl.ds(base, 16)], v)

    gather(0).start()
    for gi in range(_NCH):
        if gi + 1 < _NCH:
            gather(gi + 1).start()
        gather(gi).wait()
        accum_chunk(bufs[gi % 2], first=(gi == 0))

    def prow(k, _):
        rr = sid * _RPT + k
        par = (rr % 2).astype(jnp.float32)
        patt = pat1_f + (pat3_f - pat1_f) * par

        @plsc.parallel_loop(0, _NV, unroll=4)
        def pcol(j):
            base = j * 16
            hm = acc[k, pl.ds(base, 16)] * inv_c
            p_buf[pl.ds(k * _W + base, 16)] = jnp.where(hm <= _THR, patt, zero)
        return 0
    lax.fori_loop(0, _RPT, prow, 0)

    pltpu.sync_copy(p_buf.at[pl.ds(0, _W)], shared.at[sid])
    plsc.subcore_barrier()

    @pl.when(sid < _NSUB - 1)
    def _():
        pltpu.sync_copy(shared.at[sid + 1], p_buf.at[pl.ds(_RPT * _W, _W)])

    @pl.when(sid == _NSUB - 1)
    def _():
        @plsc.parallel_loop(0, _NV, unroll=4)
        def zcol(j):
            p_buf[pl.ds(_RPT * _W + j * 16, 16)] = zero

    def orow(k, _):
        def ocol(j, carry):
            prev, cur = carry
            base = j * 16
            flat = k * _W + base
            last = (j == _NV - 1).astype(jnp.float32)
            nxt = p_buf[pl.ds(flat + 16, 16)] * (1.0 - last)
            pdn = p_buf[pl.ds(flat + _W, 16)]
            psr = jnp.where(io == 0, _take(prev, idx_m1), _take(cur, idx_m1))
            psl = jnp.where(io == 15, _take(nxt, idx_p1), _take(cur, idx_p1))
            mtile[2 * k, pl.ds(base, 16)] = even_f * (one - psr)
            mtile[2 * k + 1, pl.ds(base, 16)] = odd_f * (one - jnp.maximum(cur, pdn))
            vtile[2 * k, pl.ds(base, 16)] = even_f * psl
            vtile[2 * k + 1, pl.ds(base, 16)] = zero
            return (cur, nxt)
        cur0 = p_buf[pl.ds(k * _W, 16)]
        lax.fori_loop(0, _NV, ocol, (zero, cur0), unroll=2)
        return 0
    lax.fori_loop(0, _RPT, orow, 0)

    rows = pl.ds(sid * 2 * _RPT, 2 * _RPT)
    pltpu.sync_copy(mtile, mask_out.at[cid, 0, rows, :])
    pltpu.sync_copy(vtile, val_out.at[cid, 0, rows, :])


@jax.jit
def kernel(sigma):
    table = sigma.reshape(_B * _C * _H, _W)
    idxs = _build_indices()
    out_sds = jax.ShapeDtypeStruct((_B, 1, _H, _W), jnp.float32)
    mesh = plsc.VectorSubcoreMesh(core_axis_name="c", subcore_axis_name="s")
    sc_fn = functools.partial(
        pl.kernel,
        mesh=mesh,
        out_type=[out_sds, out_sds],
        scratch_types=[
            pltpu.VMEM((_NCH, _IPC), jnp.int32),
            pltpu.VMEM((_IPC, _W), jnp.float32),
            pltpu.VMEM((_IPC, _W), jnp.float32),
            pltpu.VMEM((_RPT, _W), jnp.float32),
            pltpu.VMEM(((_RPT + 1) * _W,), jnp.float32),
            pltpu.VMEM((2 * _RPT, _W), jnp.float32),
            pltpu.VMEM((2 * _RPT, _W), jnp.float32),
            pltpu.VMEM_SHARED((_NSUB, _W), jnp.float32),
            pltpu.SemaphoreType.DMA,
            pltpu.SemaphoreType.DMA,
        ],
    )(_sc_body)
    mask, values = sc_fn(table, idxs)
    return mask, values


# speedup vs baseline: 1.0940x; 1.0553x over previous
import functools

import numpy as np
import jax
import jax.numpy as jnp
from jax import lax
from jax.experimental import pallas as pl
from jax.experimental.pallas import tpu as pltpu
from jax.experimental.pallas import tpu_sc as plsc

_THR = 0.18
_B, _C, _H, _W = 2, 96, 512, 512
_HR = _H // 2
_NSUB = 16
_RPT = _HR // _NSUB
_NV = _W // 16
_ORING = 4


def _build_indices():
    b = np.arange(_B)[:, None, None, None]
    sid = np.arange(_NSUB)[None, :, None, None]
    k = np.arange(_RPT)[None, None, :, None]
    ch = np.arange(_C)[None, None, None, :]
    r = sid * _RPT + k
    row_id = (b * _C + ch) * _H + 2 * r
    return jnp.asarray(row_id.astype(np.int32))


def _sc_body(table, idxs, mask_out, val_out,
             idx_v, buf0, buf1, p_buf, om, ov, shared, sem0, sem1, semo):
    cid = lax.axis_index("c")
    sid = lax.axis_index("s")

    io = lax.iota(jnp.int32, 16)
    one = jnp.full((16,), 1.0, jnp.float32)
    zero = jnp.full((16,), 0.0, jnp.float32)
    even_f = jnp.where(io % 2 == 0, 1.0, 0.0)
    odd_f = one - even_f
    pat1_f = jnp.where(io % 4 == 1, 1.0, 0.0)
    pat3_f = jnp.where(io % 4 == 3, 1.0, 0.0)
    idx_m1 = (io + 15) % 16
    idx_p1 = (io + 1) % 16
    inv_c = jnp.float32(1.0 / _C)

    _gdn = lax.GatherDimensionNumbers(
        offset_dims=(), collapsed_slice_dims=(0,), start_index_map=(0,))

    def _take(v, idx):
        return lax.gather(v, idx[:, None], _gdn, (1,),
                          mode=lax.GatherScatterMode.PROMISE_IN_BOUNDS)

    pltpu.sync_copy(idxs.at[cid, sid], idx_v)

    bufs = (buf0, buf1)
    sems = (sem0, sem1)

    def gather(k):
        return pltpu.make_async_copy(
            table.at[idx_v.at[k]], bufs[k % 2], sems[k % 2])

    def sum_row(k):
        buf = bufs[k % 2]
        patt = pat1_f if k % 2 == 0 else pat3_f

        @plsc.parallel_loop(0, _NV, unroll=1)
        def pcol(j):
            base = j * 16
            accs = [buf[ch, pl.ds(base, 16)] for ch in range(4)]
            for ch in range(4, _C):
                accs[ch % 4] = accs[ch % 4] + buf[ch, pl.ds(base, 16)]
            v = (accs[0] + accs[1]) + (accs[2] + accs[3])
            p_buf[pl.ds(k * _W + base, 16)] = jnp.where(
                v * inv_c <= _THR, patt, zero)

    def out_copies(r):
        rows = pl.ds((sid * _RPT + r) * 2, 2)
        cm = pltpu.make_async_copy(om.at[r % _ORING], mask_out.at[cid, 0, rows, :], semo)
        cv = pltpu.make_async_copy(ov.at[r % _ORING], val_out.at[cid, 0, rows, :], semo)
        return cm, cv

    def emit_orow(r):
        if r >= _ORING:
            for cp in out_copies(r - _ORING):
                cp.wait()
        mt = om.at[r % _ORING]
        vt = ov.at[r % _ORING]

        def ocol(j, carry):
            prev, cur = carry
            base = j * 16
            flat = r * _W + base
            last = (j == _NV - 1).astype(jnp.float32)
            nxt = p_buf[pl.ds(flat + 16, 16)] * (1.0 - last)
            pdn = p_buf[pl.ds(flat + _W, 16)]
            psr = jnp.where(io == 0, _take(prev, idx_m1), _take(cur, idx_m1))
            psl = jnp.where(io == 15, _take(nxt, idx_p1), _take(cur, idx_p1))
            mt[0, pl.ds(base, 16)] = even_f * (one - psr)
            mt[1, pl.ds(base, 16)] = odd_f * (one - jnp.maximum(cur, pdn))
            vt[0, pl.ds(base, 16)] = even_f * psl
            vt[1, pl.ds(base, 16)] = zero
            return (cur, nxt)
        cur0 = p_buf[pl.ds(r * _W, 16)]
        lax.fori_loop(0, _NV, ocol, (zero, cur0), unroll=2)
        for cp in out_copies(r):
            cp.start()

    gather(0).start()
    for k in range(_RPT):
        if k + 1 < _RPT:
            gather(k + 1).start()
        gather(k).wait()
        sum_row(k)
        if k == 0:
            pltpu.sync_copy(p_buf.at[pl.ds(0, _W)], shared.at[sid])
            plsc.subcore_barrier()

            @pl.when(sid < _NSUB - 1)
            def _():
                pltpu.sync_copy(shared.at[sid + 1],
                                p_buf.at[pl.ds(_RPT * _W, _W)])

            @pl.when(sid == _NSUB - 1)
            def _():
                @plsc.parallel_loop(0, _NV, unroll=4)
                def zcol(j):
                    p_buf[pl.ds(_RPT * _W + j * 16, 16)] = zero
        if k >= 1:
            emit_orow(k - 1)
    emit_orow(_RPT - 1)
    for r in (_RPT - _ORING, _RPT - 3, _RPT - 2, _RPT - 1):
        for cp in out_copies(r):
            cp.wait()


@jax.jit
def kernel(sigma):
    table = sigma.reshape(_B * _C * _H, _W)
    idxs = _build_indices()
    out_sds = jax.ShapeDtypeStruct((_B, 1, _H, _W), jnp.float32)
    mesh = plsc.VectorSubcoreMesh(core_axis_name="c", subcore_axis_name="s")
    sc_fn = functools.partial(
        pl.kernel,
        mesh=mesh,
        out_type=[out_sds, out_sds],
        scratch_types=[
            pltpu.VMEM((_RPT, _C), jnp.int32),
            pltpu.VMEM((_C, _W), jnp.float32),
            pltpu.VMEM((_C, _W), jnp.float32),
            pltpu.VMEM(((_RPT + 1) * _W,), jnp.float32),
            pltpu.VMEM((_ORING, 2, _W), jnp.float32),
            pltpu.VMEM((_ORING, 2, _W), jnp.float32),
            pltpu.VMEM_SHARED((_NSUB, _W), jnp.float32),
            pltpu.SemaphoreType.DMA,
            pltpu.SemaphoreType.DMA,
            pltpu.SemaphoreType.DMA,
        ],
    )(_sc_body)
    mask, values = sc_fn(table, idxs)
    return mask, values
